# baseline (speedup 1.0000x reference)
import functools

import jax
import jax.numpy as jnp
from jax import lax
from jax.experimental import pallas as pl
from jax.experimental.pallas import tpu as pltpu
from jax.experimental.pallas import tpu_sc as plsc

SLOPE = (1.0 / 8.0 + 1.0 / 3.0) / 2.0

_EBLK = 80
_NPAD = 50048
_PART = 12512
_ACC = 12544
_TROWS = 784
_ECHUNK = 800
_CCAP = 960


def _sc_agg_body(x, src, dst, out, sv, dv, csrc, cdst, didx, st0, st1, zbuf,
                 acc, sem0, sem1):
    c = lax.axis_index("c")
    s = lax.axis_index("s")
    nblocks = src.shape[0] // _ECHUNK
    trips = (nblocks - s + 15) // 16

    def _zrow(i, _):
        for k in range(8):
            zbuf[i, k * 16:(k + 1) * 16] = jnp.zeros((16,), jnp.float32)
        return 0
    lax.fori_loop(0, zbuf.shape[0], _zrow, 0)

    def _cfill(i, _):
        csrc[pl.ds(i * 16, 16)] = jnp.zeros((16,), jnp.int32)
        cdst[pl.ds(i * 16, 16)] = jnp.full((16,), _PART, jnp.int32)
        return 0
    lax.fori_loop(0, _CCAP // 16, _cfill, 0)

    def _scat(kk):
        for g in range(_EBLK // 16):
            didx[pl.ds(g * 16, 16)] = cdst[pl.ds(kk * _EBLK + g * 16, 16)]

    def _flush(nb, ptr):
        def _pair(i, _):
            k0 = 2 * i
            k1 = 2 * i + 1
            h0 = pltpu.async_copy(
                x.at[csrc.at[pl.ds(k0 * _EBLK, _EBLK)]], st0, sem0)
            h1 = pltpu.async_copy(
                x.at[csrc.at[pl.ds(k1 * _EBLK, _EBLK)]], st1, sem1)
            h0.wait()
            _scat(k0)
            pltpu.sync_copy(st0, acc.at[didx], add=True)
            h1.wait()

            @pl.when(k1 < nb)
            def _do1():
                _scat(k1)
                pltpu.sync_copy(st1, acc.at[didx], add=True)
            return 0
        lax.fori_loop(0, (nb + 1) // 2, _pair, 0)
        rem_base = nb * _EBLK
        for k in range(_EBLK // 16):
            vs = csrc[pl.ds(rem_base + k * 16, 16)]
            vd = cdst[pl.ds(rem_base + k * 16, 16)]
            csrc[pl.ds(k * 16, 16)] = vs
            cdst[pl.ds(k * 16, 16)] = vd
        return ptr - nb * _EBLK

    for j in (0, 1):
        base = (2 * c + j) * _PART

        for k in range(_TROWS // 8):
            pltpu.sync_copy(zbuf, acc.at[pl.ds(s * _TROWS + k * 8, 8)])
        plsc.subcore_barrier()

        def _chunk(i, ptr):
            off = pl.multiple_of((s + i * 16) * _ECHUNK, 8)
            pltpu.sync_copy(src.at[pl.ds(off, _ECHUNK)], sv)
            pltpu.sync_copy(dst.at[pl.ds(off, _ECHUNK)], dv)
            ptr_vec = jnp.full((16,), ptr, jnp.int32)

            def _compact(g, pv):
                d = dv[pl.ds(g * 16, 16)]
                sr = sv[pl.ds(g * 16, 16)]
                dl = d - base
                m = (dl >= 0) & (dl < _PART)
                sdl, ssr, om = plsc.sort_key_val(dl, sr, mask=m)
                pos = pv + lax.iota(jnp.int32, 16)
                plsc.store_scatter(cdst, [pos], sdl, mask=om)
                plsc.store_scatter(csrc, [pos], ssr, mask=om)
                return pv + plsc.all_reduce_population_count(m)
            ptr_vec = lax.fori_loop(0, _ECHUNK // 16, _compact, ptr_vec)
            ptr = ptr_vec[0]
            return _flush(ptr // _EBLK, ptr)
        ptr = lax.fori_loop(0, trips, _chunk, jnp.int32(0))

        for k in range(_EBLK // 16):
            cdst[pl.ds(ptr + k * 16, 16)] = jnp.full((16,), _PART, jnp.int32)
            csrc[pl.ds(ptr + k * 16, 16)] = jnp.zeros((16,), jnp.int32)
        _flush((ptr + _EBLK - 1) // _EBLK, ptr)
        plsc.subcore_barrier()

        part_off = pl.multiple_of(base + s * _TROWS, 8)

        @pl.when(s < 15)
        def _wb_full():
            pltpu.sync_copy(acc.at[pl.ds(s * _TROWS, _TROWS)],
                            out.at[pl.ds(part_off, _TROWS)])

        @pl.when(s == 15)
        def _wb_last():
            rem = _PART - 15 * _TROWS
            pltpu.sync_copy(acc.at[pl.ds(15 * _TROWS, rem)],
                            out.at[pl.ds(part_off, rem)])
        plsc.subcore_barrier()


def _sc_agg(x, src, dst):
    mesh = plsc.VectorSubcoreMesh(core_axis_name="c", subcore_axis_name="s")
    f = pl.kernel(
        _sc_agg_body, mesh=mesh,
        compiler_params=pltpu.CompilerParams(needs_layout_passes=False),
        out_type=jax.ShapeDtypeStruct((_NPAD, 128), jnp.float32),
        scratch_types=[
            pltpu.VMEM((_ECHUNK,), jnp.int32),
            pltpu.VMEM((_ECHUNK,), jnp.int32),
            pltpu.VMEM((_CCAP,), jnp.int32),
            pltpu.VMEM((_CCAP,), jnp.int32),
            pltpu.VMEM((_EBLK,), jnp.int32),
            pltpu.VMEM((_EBLK, 128), jnp.float32),
            pltpu.VMEM((_EBLK, 128), jnp.float32),
            pltpu.VMEM((8, 128), jnp.float32),
            pltpu.VMEM_SHARED((_ACC, 128), jnp.float32),
            pltpu.SemaphoreType.DMA,
            pltpu.SemaphoreType.DMA,
        ],
    )
    return f(x, src, dst)


def _conv_body(agg0, cnt0, agg1, cnt1, xin, wl0, wl1, wr, bias, out):
    c0 = jnp.maximum(cnt0[...], 1.0)
    c1 = jnp.maximum(cnt1[...], 1.0)
    m0 = agg0[...] / c0
    m1 = agg1[...] / c1
    acc = jnp.dot(m0, wl0[...], preferred_element_type=jnp.float32)
    acc = acc + jnp.dot(m1, wl1[...], preferred_element_type=jnp.float32)
    acc = acc + jnp.dot(xin[...], wr[...], preferred_element_type=jnp.float32)
    acc = (acc + bias[...]) * 0.5
    out[...] = jnp.where(acc >= 0, acc, SLOPE * acc)


def _conv(agg0, cnt0, agg1, cnt1, xin, Wl0T, Wl1T, WrT, bias, bn=2000):
    n, d = xin.shape
    h = Wl0T.shape[1]
    aspec = pl.BlockSpec((bn, d), lambda i: (i, 0))
    cspec = pl.BlockSpec((bn, 1), lambda i: (i, 0))
    wspec = pl.BlockSpec((d, h), lambda i: (0, 0))
    return pl.pallas_call(
        _conv_body,
        grid=(n // bn,),
        in_specs=[aspec, cspec, aspec, cspec, aspec,
                  wspec, wspec, wspec,
                  pl.BlockSpec((1, h), lambda i: (0, 0))],
        out_specs=pl.BlockSpec((bn, h), lambda i: (i, 0)),
        out_shape=jax.ShapeDtypeStruct((n, h), jnp.float32),
    )(agg0, cnt0, agg1, cnt1, xin, Wl0T, Wl1T, WrT, bias)


def _relrepr_body(cnt0, cnt1, cd2, rel_emb, wlT_a, wlT_b, blinr, wcrT, bcr,
                  num_edges, out_R, out_conr, acc):
    i = pl.program_id(0)

    @pl.when(i == 0)
    def _init():
        acc[...] = jnp.zeros_like(acc)

    blk = cd2[...]
    a0 = jnp.sum(cnt0[...] * blk, axis=0, keepdims=True)
    a1 = jnp.sum(cnt1[...] * blk, axis=0, keepdims=True)
    acc[0:1, :] += a0
    acc[1:2, :] += a1

    @pl.when(i == pl.num_programs(0) - 1)
    def _fin():
        avg = acc[0:2, :] * (1.0 / num_edges)
        cur = (jnp.dot(avg, wlT_a[...], preferred_element_type=jnp.float32)
               + jnp.dot(rel_emb[...], wlT_b[...],
                         preferred_element_type=jnp.float32)
               + blinr[...])
        out_R[...] = cur
        out_conr[...] = (jnp.dot(cur, wcrT[...],
                                 preferred_element_type=jnp.float32)
                         + bcr[...])


def _relrepr(cnt0, cnt1, cd2, rel_emb, Wlinr, blinr, Wcr, bcr, num_edges,
             bn=2000):
    n, h = cd2.shape
    oc = Wcr.shape[0]
    wlT_a = Wlinr[:, :h].T
    wlT_b = Wlinr[:, h:].T
    return pl.pallas_call(
        lambda *a: _relrepr_body(*a[:9], num_edges, *a[9:]),
        grid=(n // bn,),
        in_specs=[
            pl.BlockSpec((bn, 1), lambda i: (i, 0)),
            pl.BlockSpec((bn, 1), lambda i: (i, 0)),
            pl.BlockSpec((bn, h), lambda i: (i, 0)),
            pl.BlockSpec((2, h), lambda i: (0, 0)),
            pl.BlockSpec((h, h), lambda i: (0, 0)),
            pl.BlockSpec((h, h), lambda i: (0, 0)),
            pl.BlockSpec((1, h), lambda i: (0, 0)),
            pl.BlockSpec((h, oc), lambda i: (0, 0)),
            pl.BlockSpec((1, oc), lambda i: (0, 0)),
        ],
        out_specs=[
            pl.BlockSpec((2, h), lambda i: (0, 0)),
            pl.BlockSpec((2, oc), lambda i: (0, 0)),
        ],
        out_shape=[
            jax.ShapeDtypeStruct((2, h), jnp.float32),
            jax.ShapeDtypeStruct((2, oc), jnp.float32),
        ],
        scratch_shapes=[pltpu.VMEM((8, h), jnp.float32)],
    )(cnt0, cnt1, cd2, rel_emb, wlT_a, wlT_b, blinr.reshape(1, h), Wcr.T,
      bcr.reshape(1, oc))


def _scoreB_body(w, b, a0, a1, a2, a3, o0, o1, o2, o3):
    wv = w[...]
    bv = b[...]
    o0[...] = jnp.dot(a0[...], wv, preferred_element_type=jnp.float32) + bv
    o1[...] = jnp.dot(a1[...], wv, preferred_element_type=jnp.float32) + bv
    o2[...] = jnp.dot(a2[...], wv, preferred_element_type=jnp.float32) + bv
    o3[...] = jnp.dot(a3[...], wv, preferred_element_type=jnp.float32) + bv


def _scoreB(Wch, bch, at_list, bc=2000):
    oc, h = Wch.shape
    n = at_list[0].shape[0]
    outs = pl.pallas_call(
        _scoreB_body,
        grid=(n // bc,),
        in_specs=[
            pl.BlockSpec((h, oc), lambda i: (0, 0)),
            pl.BlockSpec((1, oc), lambda i: (0, 0)),
        ] + [pl.BlockSpec((bc, h), lambda i: (i, 0))] * 4,
        out_specs=[pl.BlockSpec((bc, oc), lambda i: (i, 0))] * 4,
        out_shape=[jax.ShapeDtypeStruct((n, oc), jnp.float32)] * 4,
    )(Wch.T, bch.reshape(1, oc), *at_list)
    return outs


def _scoreH_body(ch0, ct0, ch1, ct1, w0, w2, conr, w1, bsum, h0, h1):
    consts = (jnp.dot(conr[...], w1[...], preferred_element_type=jnp.float32)
              + bsum[...])
    v0 = w0[...]
    v2 = w2[...]
    h0[...] = (jnp.dot(ch0[...], v0, preferred_element_type=jnp.float32)
               + jnp.dot(ct0[...], v2, preferred_element_type=jnp.float32)
               + consts[0:1, 0:1])
    h1[...] = (jnp.dot(ch1[...], v0, preferred_element_type=jnp.float32)
               + jnp.dot(ct1[...], v2, preferred_element_type=jnp.float32)
               + consts[1:2, 0:1])


def _scoreH(c_list, w0, w2, conr, w1, bsum, bl=2000):
    n, oc = c_list[0].shape
    outs = pl.pallas_call(
        _scoreH_body,
        grid=(n // bl,),
        in_specs=[pl.BlockSpec((bl, oc), lambda i: (i, 0))] * 4 + [
            pl.BlockSpec((oc, 1), lambda i: (0, 0)),
            pl.BlockSpec((oc, 1), lambda i: (0, 0)),
            pl.BlockSpec((2, oc), lambda i: (0, 0)),
            pl.BlockSpec((oc, 1), lambda i: (0, 0)),
            pl.BlockSpec((1, 1), lambda i: (0, 0)),
        ],
        out_specs=[pl.BlockSpec((bl, 1), lambda i: (i, 0))] * 2,
        out_shape=[jax.ShapeDtypeStruct((n, 1), jnp.float32)] * 2,
    )(*c_list, w0, w2, conr, w1, bsum)
    return outs


def _cnt_xla(dst, n):
    return jnp.zeros((n, 1), jnp.float32).at[dst, 0].add(1.0)


_GROWS = 6256


def _sc_gather_body(table, idx, out, iv, st0, st1, sem0, sem1):
    c = lax.axis_index("c")
    s = lax.axis_index("s")
    w = c * 16 + s
    rbase = pl.multiple_of(w * _GROWS, 8)
    pltpu.sync_copy(idx.at[pl.ds(rbase, _GROWS)], iv)

    nfull = _GROWS // _EBLK
    rem = _GROWS - nfull * _EBLK
    nblk = nfull + (1 if rem else 0)
    sts = (st0, st1)
    sems = (sem0, sem1)

    def _start(k):
        n = _EBLK if k < nfull else rem
        return pltpu.async_copy(
            table.at[iv.at[pl.ds(k * _EBLK, n)]],
            sts[k % 2].at[pl.ds(0, n)], sems[k % 2])

    h = _start(0)
    for k in range(nblk):
        h.wait()
        if k + 1 < nblk:
            h = _start(k + 1)
        n = _EBLK if k < nfull else rem
        pltpu.sync_copy(sts[k % 2].at[pl.ds(0, n)],
                        out.at[pl.ds(rbase + k * _EBLK, n)])


def _sc_gather(table, idxpad):
    nrows = idxpad.shape[0]
    mesh = plsc.VectorSubcoreMesh(core_axis_name="c", subcore_axis_name="s")
    f = pl.kernel(
        _sc_gather_body, mesh=mesh,
        compiler_params=pltpu.CompilerParams(needs_layout_passes=False),
        out_type=jax.ShapeDtypeStruct((nrows, 128), jnp.float32),
        scratch_types=[
            pltpu.VMEM((_GROWS,), jnp.int32),
            pltpu.VMEM((_EBLK, 128), jnp.float32),
            pltpu.VMEM((_EBLK, 128), jnp.float32),
            pltpu.SemaphoreType.DMA,
            pltpu.SemaphoreType.DMA,
        ],
    )
    return f(table, idxpad)


def kernel(x_node, edge_index_r0, edge_index_r1, edge_label_index_r0,
           edge_label_index_r1, snap, W1l_r0, b1l_r0, W1r_r0, W1l_r1, b1l_r1,
           W1r_r1, W2l_r0, b2l_r0, W2r_r0, W2l_r1, b2l_r1, W2r_r1, rel_emb,
           Wlinr, blinr, Wch, bch, Wcr, bcr, Wpost, bpost):
    n, d = x_node.shape
    h2 = W2l_r0.shape[0]
    oc = Wch.shape[0]
    num_edges = edge_index_r0.shape[1]
    l = edge_label_index_r0.shape[1]

    src0, dst0 = edge_index_r0[0], edge_index_r0[1]
    src1, dst1 = edge_index_r1[0], edge_index_r1[1]
    cnt0 = _cnt_xla(dst0, n)
    cnt1 = _cnt_xla(dst1, n)

    agg0 = _sc_agg(x_node, src0, dst0)
    agg1 = _sc_agg(x_node, src1, dst1)
    cd1 = _conv(agg0, cnt0, agg1, cnt1, x_node,
                W1l_r0.T, W1l_r1.T, (W1r_r0 + W1r_r1).T,
                (b1l_r0 + b1l_r1).reshape(1, -1))
    agg0b = _sc_agg(cd1, src0, dst0)
    agg1b = _sc_agg(cd1, src1, dst1)
    cd2 = _conv(agg0b, cnt0, agg1b, cnt1, cd1,
                W2l_r0.T, W2l_r1.T, (W2r_r0 + W2r_r1).T,
                (b2l_r0 + b2l_r1).reshape(1, -1))

    current_R, conr = _relrepr(cnt0, cnt1, cd2, rel_emb, Wlinr, blinr,
                               Wcr, bcr, float(num_edges))

    a_h0 = cd2[edge_label_index_r0[0]].reshape(h2, l).T
    a_t0 = cd2[edge_label_index_r0[1]].reshape(h2, l).T
    a_h1 = cd2[edge_label_index_r1[0]].reshape(h2, l).T
    a_t1 = cd2[edge_label_index_r1[1]].reshape(h2, l).T
    bt_list = _scoreB(Wch, bch, [a_h0, a_t0, a_h1, a_t1])
    c_list = [bt.T.reshape(l, oc) for bt in bt_list]

    w = Wpost[0] + Wpost[1]
    w0 = w[:oc].reshape(oc, 1)
    w1 = w[oc:2 * oc].reshape(oc, 1)
    w2 = w[2 * oc:].reshape(oc, 1)
    bsum = (bpost[0] + bpost[1]).reshape(1, 1)
    h0, h1 = _scoreH(c_list, w0, w2, conr, w1, bsum)
    return (h0.reshape(l), h1.reshape(l), cd1, cd2, current_R)


# speedup vs baseline: 2.1898x; 2.1898x over previous
import functools

import jax
import jax.numpy as jnp
from jax import lax
from jax.experimental import pallas as pl
from jax.experimental.pallas import tpu as pltpu
from jax.experimental.pallas import tpu_sc as plsc

SLOPE = (1.0 / 8.0 + 1.0 / 3.0) / 2.0

_EBLK = 96
_NPAD = 50048
_PART = 12512
_ACC = 12544
_TROWS = 784
_ECHUNK = 800
_CCAP = 1152


def _sc_agg_body(x, src, dst, out, sv, dv, csrc, cdst, didx, stage, zbuf,
                 acc):
    c = lax.axis_index("c")
    s = lax.axis_index("s")
    nblocks = src.shape[0] // _ECHUNK
    trips = (nblocks - s + 15) // 16

    def _zrow(i, _):
        for k in range(8):
            zbuf[i, k * 16:(k + 1) * 16] = jnp.zeros((16,), jnp.float32)
        return 0
    lax.fori_loop(0, zbuf.shape[0], _zrow, 0)

    def _flush(nb, ptr):
        def _edge(k, _):
            pltpu.sync_copy(x.at[csrc.at[pl.ds(k * _EBLK, _EBLK)]], stage)
            for g in range(_EBLK // 16):
                didx[pl.ds(g * 16, 16)] = cdst[pl.ds(k * _EBLK + g * 16, 16)]
            pltpu.sync_copy(stage, acc.at[didx], add=True)
            return 0
        lax.fori_loop(0, nb, _edge, 0)
        rem_base = nb * _EBLK
        for k in range(_EBLK // 16):
            vs = csrc[pl.ds(rem_base + k * 16, 16)]
            vd = cdst[pl.ds(rem_base + k * 16, 16)]
            csrc[pl.ds(k * 16, 16)] = vs
            cdst[pl.ds(k * 16, 16)] = vd
        return ptr - nb * _EBLK

    for j in (0, 1):
        base = (2 * c + j) * _PART

        for k in range(_TROWS // 56):
            pltpu.sync_copy(zbuf, acc.at[pl.ds(s * _TROWS + k * 56, 56)])
        plsc.subcore_barrier()

        def _chunk(i, ptr):
            off = pl.multiple_of((s + i * 16) * _ECHUNK, 8)
            pltpu.sync_copy(src.at[pl.ds(off, _ECHUNK)], sv)
            pltpu.sync_copy(dst.at[pl.ds(off, _ECHUNK)], dv)
            ptr_vec = jnp.full((16,), ptr, jnp.int32)

            def _compact(g, pv):
                d = dv[pl.ds(g * 16, 16)]
                sr = sv[pl.ds(g * 16, 16)]
                dl = d - base
                m = (dl >= 0) & (dl < _PART)
                sdl, ssr, om = plsc.sort_key_val(dl, sr, mask=m)
                pos = pv + lax.iota(jnp.int32, 16)
                plsc.store_scatter(cdst, [pos], sdl, mask=om)
                plsc.store_scatter(csrc, [pos], ssr, mask=om)
                return pv + plsc.all_reduce_population_count(m)
            ptr_vec = lax.fori_loop(0, _ECHUNK // 16, _compact, ptr_vec)
            ptr = ptr_vec[0]
            return _flush(ptr // _EBLK, ptr)
        ptr = lax.fori_loop(0, trips, _chunk, jnp.int32(0))

        for k in range(_EBLK // 16):
            cdst[pl.ds(ptr + k * 16, 16)] = jnp.full((16,), _PART, jnp.int32)
            csrc[pl.ds(ptr + k * 16, 16)] = jnp.zeros((16,), jnp.int32)
        _flush((ptr + _EBLK - 1) // _EBLK, ptr)
        plsc.subcore_barrier()

        part_off = pl.multiple_of(base + s * _TROWS, 8)

        @pl.when(s < 15)
        def _wb_full():
            pltpu.sync_copy(acc.at[pl.ds(s * _TROWS, _TROWS)],
                            out.at[pl.ds(part_off, _TROWS)])

        @pl.when(s == 15)
        def _wb_last():
            rem = _PART - 15 * _TROWS
            pltpu.sync_copy(acc.at[pl.ds(15 * _TROWS, rem)],
                            out.at[pl.ds(part_off, rem)])
        plsc.subcore_barrier()


def _sc_agg(x, src, dst):
    mesh = plsc.VectorSubcoreMesh(core_axis_name="c", subcore_axis_name="s")
    f = pl.kernel(
        _sc_agg_body, mesh=mesh,
        compiler_params=pltpu.CompilerParams(needs_layout_passes=False),
        out_type=jax.ShapeDtypeStruct((_NPAD, 128), jnp.float32),
        scratch_types=[
            pltpu.VMEM((_ECHUNK,), jnp.int32),
            pltpu.VMEM((_ECHUNK,), jnp.int32),
            pltpu.VMEM((_CCAP,), jnp.int32),
            pltpu.VMEM((_CCAP,), jnp.int32),
            pltpu.VMEM((_EBLK,), jnp.int32),
            pltpu.VMEM((_EBLK, 128), jnp.float32),
            pltpu.VMEM((56, 128), jnp.float32),
            pltpu.VMEM_SHARED((_ACC, 128), jnp.float32),
        ],
    )
    return f(x, src, dst)


def _conv_body(agg0, cnt0, agg1, cnt1, xin, wl0, wl1, wr, bias, out):
    c0 = jnp.maximum(cnt0[...], 1.0)
    c1 = jnp.maximum(cnt1[...], 1.0)
    m0 = agg0[...] / c0
    m1 = agg1[...] / c1
    acc = jnp.dot(m0, wl0[...], preferred_element_type=jnp.float32)
    acc = acc + jnp.dot(m1, wl1[...], preferred_element_type=jnp.float32)
    acc = acc + jnp.dot(xin[...], wr[...], preferred_element_type=jnp.float32)
    acc = (acc + bias[...]) * 0.5
    out[...] = jnp.where(acc >= 0, acc, SLOPE * acc)


def _conv(agg0, cnt0, agg1, cnt1, xin, Wl0T, Wl1T, WrT, bias, bn=2000):
    n, d = xin.shape
    h = Wl0T.shape[1]
    aspec = pl.BlockSpec((bn, d), lambda i: (i, 0))
    cspec = pl.BlockSpec((bn, 1), lambda i: (i, 0))
    wspec = pl.BlockSpec((d, h), lambda i: (0, 0))
    return pl.pallas_call(
        _conv_body,
        grid=(n // bn,),
        in_specs=[aspec, cspec, aspec, cspec, aspec,
                  wspec, wspec, wspec,
                  pl.BlockSpec((1, h), lambda i: (0, 0))],
        out_specs=pl.BlockSpec((bn, h), lambda i: (i, 0)),
        out_shape=jax.ShapeDtypeStruct((n, h), jnp.float32),
    )(agg0, cnt0, agg1, cnt1, xin, Wl0T, Wl1T, WrT, bias)


def _relrepr_body(cnt0, cnt1, cd2, rel_emb, wlT_a, wlT_b, blinr, wcrT, bcr,
                  num_edges, out_R, out_conr, acc):
    i = pl.program_id(0)

    @pl.when(i == 0)
    def _init():
        acc[...] = jnp.zeros_like(acc)

    blk = cd2[...]
    a0 = jnp.sum(cnt0[...] * blk, axis=0, keepdims=True)
    a1 = jnp.sum(cnt1[...] * blk, axis=0, keepdims=True)
    acc[0:1, :] += a0
    acc[1:2, :] += a1

    @pl.when(i == pl.num_programs(0) - 1)
    def _fin():
        avg = acc[0:2, :] * (1.0 / num_edges)
        cur = (jnp.dot(avg, wlT_a[...], preferred_element_type=jnp.float32)
               + jnp.dot(rel_emb[...], wlT_b[...],
                         preferred_element_type=jnp.float32)
               + blinr[...])
        out_R[...] = cur
        out_conr[...] = (jnp.dot(cur, wcrT[...],
                                 preferred_element_type=jnp.float32)
                         + bcr[...])


def _relrepr(cnt0, cnt1, cd2, rel_emb, Wlinr, blinr, Wcr, bcr, num_edges,
             bn=2000):
    n, h = cd2.shape
    oc = Wcr.shape[0]
    wlT_a = Wlinr[:, :h].T
    wlT_b = Wlinr[:, h:].T
    return pl.pallas_call(
        lambda *a: _relrepr_body(*a[:9], num_edges, *a[9:]),
        grid=(n // bn,),
        in_specs=[
            pl.BlockSpec((bn, 1), lambda i: (i, 0)),
            pl.BlockSpec((bn, 1), lambda i: (i, 0)),
            pl.BlockSpec((bn, h), lambda i: (i, 0)),
            pl.BlockSpec((2, h), lambda i: (0, 0)),
            pl.BlockSpec((h, h), lambda i: (0, 0)),
            pl.BlockSpec((h, h), lambda i: (0, 0)),
            pl.BlockSpec((1, h), lambda i: (0, 0)),
            pl.BlockSpec((h, oc), lambda i: (0, 0)),
            pl.BlockSpec((1, oc), lambda i: (0, 0)),
        ],
        out_specs=[
            pl.BlockSpec((2, h), lambda i: (0, 0)),
            pl.BlockSpec((2, oc), lambda i: (0, 0)),
        ],
        out_shape=[
            jax.ShapeDtypeStruct((2, h), jnp.float32),
            jax.ShapeDtypeStruct((2, oc), jnp.float32),
        ],
        scratch_shapes=[pltpu.VMEM((8, h), jnp.float32)],
    )(cnt0, cnt1, cd2, rel_emb, wlT_a, wlT_b, blinr.reshape(1, h), Wcr.T,
      bcr.reshape(1, oc))


def _scoreB_body(w, b, a0, a1, a2, a3, o0, o1, o2, o3):
    wv = w[...]
    bv = b[...]
    o0[...] = jnp.dot(a0[...], wv, preferred_element_type=jnp.float32) + bv
    o1[...] = jnp.dot(a1[...], wv, preferred_element_type=jnp.float32) + bv
    o2[...] = jnp.dot(a2[...], wv, preferred_element_type=jnp.float32) + bv
    o3[...] = jnp.dot(a3[...], wv, preferred_element_type=jnp.float32) + bv


def _scoreB(Wch, bch, at_list, bc=2000):
    oc, h = Wch.shape
    n = at_list[0].shape[0]
    outs = pl.pallas_call(
        _scoreB_body,
        grid=(n // bc,),
        in_specs=[
            pl.BlockSpec((h, oc), lambda i: (0, 0)),
            pl.BlockSpec((1, oc), lambda i: (0, 0)),
        ] + [pl.BlockSpec((bc, h), lambda i: (i, 0))] * 4,
        out_specs=[pl.BlockSpec((bc, oc), lambda i: (i, 0))] * 4,
        out_shape=[jax.ShapeDtypeStruct((n, oc), jnp.float32)] * 4,
    )(Wch.T, bch.reshape(1, oc), *at_list)
    return outs


def _scoreH_body(ch0, ct0, ch1, ct1, w0, w2, conr, w1, bsum, h0, h1):
    consts = (jnp.dot(conr[...], w1[...], preferred_element_type=jnp.float32)
              + bsum[...])
    v0 = w0[...]
    v2 = w2[...]
    h0[...] = (jnp.dot(ch0[...], v0, preferred_element_type=jnp.float32)
               + jnp.dot(ct0[...], v2, preferred_element_type=jnp.float32)
               + consts[0:1, 0:1])
    h1[...] = (jnp.dot(ch1[...], v0, preferred_element_type=jnp.float32)
               + jnp.dot(ct1[...], v2, preferred_element_type=jnp.float32)
               + consts[1:2, 0:1])


def _scoreH(c_list, w0, w2, conr, w1, bsum, bl=2000):
    n, oc = c_list[0].shape
    outs = pl.pallas_call(
        _scoreH_body,
        grid=(n // bl,),
        in_specs=[pl.BlockSpec((bl, oc), lambda i: (i, 0))] * 4 + [
            pl.BlockSpec((oc, 1), lambda i: (0, 0)),
            pl.BlockSpec((oc, 1), lambda i: (0, 0)),
            pl.BlockSpec((2, oc), lambda i: (0, 0)),
            pl.BlockSpec((oc, 1), lambda i: (0, 0)),
            pl.BlockSpec((1, 1), lambda i: (0, 0)),
        ],
        out_specs=[pl.BlockSpec((bl, 1), lambda i: (i, 0))] * 2,
        out_shape=[jax.ShapeDtypeStruct((n, 1), jnp.float32)] * 2,
    )(*c_list, w0, w2, conr, w1, bsum)
    return outs


def _cnt_xla(dst, n):
    return jnp.zeros((n, 1), jnp.float32).at[dst, 0].add(1.0)


def _conv_body(agg0, cnt0, agg1, cnt1, xin, wl0, wl1, wr, bias, out):
    c0 = jnp.maximum(cnt0[...], 1.0)
    c1 = jnp.maximum(cnt1[...], 1.0)
    m0 = agg0[...] / c0
    m1 = agg1[...] / c1
    acc = jnp.dot(m0, wl0[...], preferred_element_type=jnp.float32)
    acc = acc + jnp.dot(m1, wl1[...], preferred_element_type=jnp.float32)
    acc = acc + jnp.dot(xin[...], wr[...], preferred_element_type=jnp.float32)
    acc = (acc + bias[...]) * 0.5
    out[...] = jnp.where(acc >= 0, acc, SLOPE * acc)


def _conv(agg0, cnt0, agg1, cnt1, xin, Wl0T, Wl1T, WrT, bias, bn=2000):
    n, d = xin.shape
    h = Wl0T.shape[1]
    aspec = pl.BlockSpec((bn, d), lambda i: (i, 0))
    cspec = pl.BlockSpec((bn, 1), lambda i: (i, 0))
    wspec = pl.BlockSpec((d, h), lambda i: (0, 0))
    return pl.pallas_call(
        _conv_body,
        grid=(n // bn,),
        in_specs=[aspec, cspec, aspec, cspec, aspec,
                  wspec, wspec, wspec,
                  pl.BlockSpec((1, h), lambda i: (0, 0))],
        out_specs=pl.BlockSpec((bn, h), lambda i: (i, 0)),
        out_shape=jax.ShapeDtypeStruct((n, h), jnp.float32),
    )(agg0, cnt0, agg1, cnt1, xin, Wl0T, Wl1T, WrT, bias)


def _relrepr_body(cnt0, cnt1, cd2, rel_emb, wlT_a, wlT_b, blinr, wcrT, bcr,
                  num_edges, out_R, out_conr, acc):
    i = pl.program_id(0)

    @pl.when(i == 0)
    def _init():
        acc[...] = jnp.zeros_like(acc)

    blk = cd2[...]
    a0 = jnp.sum(cnt0[...] * blk, axis=0, keepdims=True)
    a1 = jnp.sum(cnt1[...] * blk, axis=0, keepdims=True)
    acc[0:1, :] += a0
    acc[1:2, :] += a1

    @pl.when(i == pl.num_programs(0) - 1)
    def _fin():
        avg = acc[0:2, :] * (1.0 / num_edges)
        cur = (jnp.dot(avg, wlT_a[...], preferred_element_type=jnp.float32)
               + jnp.dot(rel_emb[...], wlT_b[...],
                         preferred_element_type=jnp.float32)
               + blinr[...])
        out_R[...] = cur
        out_conr[...] = (jnp.dot(cur, wcrT[...],
                                 preferred_element_type=jnp.float32)
                         + bcr[...])


def _relrepr(cnt0, cnt1, cd2, rel_emb, Wlinr, blinr, Wcr, bcr, num_edges,
             bn=2000):
    n, h = cd2.shape
    oc = Wcr.shape[0]
    wlT_a = Wlinr[:, :h].T
    wlT_b = Wlinr[:, h:].T
    return pl.pallas_call(
        lambda *a: _relrepr_body(*a[:9], num_edges, *a[9:]),
        grid=(n // bn,),
        in_specs=[
            pl.BlockSpec((bn, 1), lambda i: (i, 0)),
            pl.BlockSpec((bn, 1), lambda i: (i, 0)),
            pl.BlockSpec((bn, h), lambda i: (i, 0)),
            pl.BlockSpec((2, h), lambda i: (0, 0)),
            pl.BlockSpec((h, h), lambda i: (0, 0)),
            pl.BlockSpec((h, h), lambda i: (0, 0)),
            pl.BlockSpec((1, h), lambda i: (0, 0)),
            pl.BlockSpec((h, oc), lambda i: (0, 0)),
            pl.BlockSpec((1, oc), lambda i: (0, 0)),
        ],
        out_specs=[
            pl.BlockSpec((2, h), lambda i: (0, 0)),
            pl.BlockSpec((2, oc), lambda i: (0, 0)),
        ],
        out_shape=[
            jax.ShapeDtypeStruct((2, h), jnp.float32),
            jax.ShapeDtypeStruct((2, oc), jnp.float32),
        ],
        scratch_shapes=[pltpu.VMEM((8, h), jnp.float32)],
    )(cnt0, cnt1, cd2, rel_emb, wlT_a, wlT_b, blinr.reshape(1, h), Wcr.T,
      bcr.reshape(1, oc))


def _scoreB_body(w, b, a0, a1, a2, a3, o0, o1, o2, o3):
    wv = w[...]
    bv = b[...]
    o0[...] = jnp.dot(a0[...], wv, preferred_element_type=jnp.float32) + bv
    o1[...] = jnp.dot(a1[...], wv, preferred_element_type=jnp.float32) + bv
    o2[...] = jnp.dot(a2[...], wv, preferred_element_type=jnp.float32) + bv
    o3[...] = jnp.dot(a3[...], wv, preferred_element_type=jnp.float32) + bv


def _scoreB(Wch, bch, at_list, bc=2000):
    oc, h = Wch.shape
    n = at_list[0].shape[0]
    outs = pl.pallas_call(
        _scoreB_body,
        grid=(n // bc,),
        in_specs=[
            pl.BlockSpec((h, oc), lambda i: (0, 0)),
            pl.BlockSpec((1, oc), lambda i: (0, 0)),
        ] + [pl.BlockSpec((bc, h), lambda i: (i, 0))] * 4,
        out_specs=[pl.BlockSpec((bc, oc), lambda i: (i, 0))] * 4,
        out_shape=[jax.ShapeDtypeStruct((n, oc), jnp.float32)] * 4,
    )(Wch.T, bch.reshape(1, oc), *at_list)
    return outs


def _scoreH_body(ch0, ct0, ch1, ct1, w0, w2, conr, w1, bsum, h0, h1):
    consts = (jnp.dot(conr[...], w1[...], preferred_element_type=jnp.float32)
              + bsum[...])
    v0 = w0[...]
    v2 = w2[...]
    h0[...] = (jnp.dot(ch0[...], v0, preferred_element_type=jnp.float32)
               + jnp.dot(ct0[...], v2, preferred_element_type=jnp.float32)
               + consts[0:1, 0:1])
    h1[...] = (jnp.dot(ch1[...], v0, preferred_element_type=jnp.float32)
               + jnp.dot(ct1[...], v2, preferred_element_type=jnp.float32)
               + consts[1:2, 0:1])


def _scoreH(c_list, w0, w2, conr, w1, bsum, bl=2000):
    n, oc = c_list[0].shape
    outs = pl.pallas_call(
        _scoreH_body,
        grid=(n // bl,),
        in_specs=[pl.BlockSpec((bl, oc), lambda i: (i, 0))] * 4 + [
            pl.BlockSpec((oc, 1), lambda i: (0, 0)),
            pl.BlockSpec((oc, 1), lambda i: (0, 0)),
            pl.BlockSpec((2, oc), lambda i: (0, 0)),
            pl.BlockSpec((oc, 1), lambda i: (0, 0)),
            pl.BlockSpec((1, 1), lambda i: (0, 0)),
        ],
        out_specs=[pl.BlockSpec((bl, 1), lambda i: (i, 0))] * 2,
        out_shape=[jax.ShapeDtypeStruct((n, 1), jnp.float32)] * 2,
    )(*c_list, w0, w2, conr, w1, bsum)
    return outs


def _cnt_xla(dst, n):
    return jnp.zeros((n, 1), jnp.float32).at[dst, 0].add(1.0)


_GROWS = 6256


def _sc_gather_body(table, idx, out, iv, st0, st1, sem0, sem1):
    c = lax.axis_index("c")
    s = lax.axis_index("s")
    w = c * 16 + s
    rbase = pl.multiple_of(w * _GROWS, 8)
    pltpu.sync_copy(idx.at[pl.ds(rbase, _GROWS)], iv)

    nfull = _GROWS // _EBLK
    rem = _GROWS - nfull * _EBLK
    nblk = nfull + (1 if rem else 0)
    sts = (st0, st1)
    sems = (sem0, sem1)

    def _start(k):
        n = _EBLK if k < nfull else rem
        return pltpu.async_copy(
            table.at[iv.at[pl.ds(k * _EBLK, n)]],
            sts[k % 2].at[pl.ds(0, n)], sems[k % 2])

    h = _start(0)
    for k in range(nblk):
        h.wait()
        if k + 1 < nblk:
            h = _start(k + 1)
        n = _EBLK if k < nfull else rem
        pltpu.sync_copy(sts[k % 2].at[pl.ds(0, n)],
                        out.at[pl.ds(rbase + k * _EBLK, n)])


def _sc_gather(table, idxpad):
    nrows = idxpad.shape[0]
    mesh = plsc.VectorSubcoreMesh(core_axis_name="c", subcore_axis_name="s")
    f = pl.kernel(
        _sc_gather_body, mesh=mesh,
        compiler_params=pltpu.CompilerParams(needs_layout_passes=False),
        out_type=jax.ShapeDtypeStruct((nrows, 128), jnp.float32),
        scratch_types=[
            pltpu.VMEM((_GROWS,), jnp.int32),
            pltpu.VMEM((_EBLK, 128), jnp.float32),
            pltpu.VMEM((_EBLK, 128), jnp.float32),
            pltpu.SemaphoreType.DMA,
            pltpu.SemaphoreType.DMA,
        ],
    )
    return f(table, idxpad)


def kernel(x_node, edge_index_r0, edge_index_r1, edge_label_index_r0,
           edge_label_index_r1, snap, W1l_r0, b1l_r0, W1r_r0, W1l_r1, b1l_r1,
           W1r_r1, W2l_r0, b2l_r0, W2r_r0, W2l_r1, b2l_r1, W2r_r1, rel_emb,
           Wlinr, blinr, Wch, bch, Wcr, bcr, Wpost, bpost):
    n, d = x_node.shape
    h2 = W2l_r0.shape[0]
    oc = Wch.shape[0]
    num_edges = edge_index_r0.shape[1]
    l = edge_label_index_r0.shape[1]

    src0, dst0 = edge_index_r0[0], edge_index_r0[1]
    src1, dst1 = edge_index_r1[0], edge_index_r1[1]
    cnt0 = _cnt_xla(dst0, n)
    cnt1 = _cnt_xla(dst1, n)

    agg0 = _sc_agg(x_node, src0, dst0)
    agg1 = _sc_agg(x_node, src1, dst1)
    cd1 = _conv(agg0, cnt0, agg1, cnt1, x_node,
                W1l_r0.T, W1l_r1.T, (W1r_r0 + W1r_r1).T,
                (b1l_r0 + b1l_r1).reshape(1, -1))
    agg0b = _sc_agg(cd1, src0, dst0)
    agg1b = _sc_agg(cd1, src1, dst1)
    cd2 = _conv(agg0b, cnt0, agg1b, cnt1, cd1,
                W2l_r0.T, W2l_r1.T, (W2r_r0 + W2r_r1).T,
                (b2l_r0 + b2l_r1).reshape(1, -1))

    current_R, conr = _relrepr(cnt0, cnt1, cd2, rel_emb, Wlinr, blinr,
                               Wcr, bcr, float(num_edges))

    a_h0 = cd2[edge_label_index_r0[0]].reshape(h2, l).T
    a_t0 = cd2[edge_label_index_r0[1]].reshape(h2, l).T
    a_h1 = cd2[edge_label_index_r1[0]].reshape(h2, l).T
    a_t1 = cd2[edge_label_index_r1[1]].reshape(h2, l).T
    bt_list = _scoreB(Wch, bch, [a_h0, a_t0, a_h1, a_t1])
    c_list = [bt.T.reshape(l, oc) for bt in bt_list]

    w = Wpost[0] + Wpost[1]
    w0 = w[:oc].reshape(oc, 1)
    w1 = w[oc:2 * oc].reshape(oc, 1)
    w2 = w[2 * oc:].reshape(oc, 1)
    bsum = (bpost[0] + bpost[1]).reshape(1, 1)
    h0, h1 = _scoreH(c_list, w0, w2, conr, w1, bsum)
    return (h0.reshape(l), h1.reshape(l), cd1, cd2, current_R)
